# Initial kernel scaffold; baseline (speedup 1.0000x reference)
import functools

import jax
import jax.numpy as jnp
from jax import lax
from jax.experimental import pallas as pl
from jax.experimental.pallas import tpu as pltpu
from jax.experimental.pallas import tpu_sc as plsc

B = 4096
F = 26
L = 50
VOCAB = 100000
D = 32

NC = 2
NS = 16
NW = NC * NS
BPW = B // NW
P = F * L
LANES = 16
PPAD = ((P + LANES - 1) // LANES) * LANES


def _body(labels_hbm, tables_hbm, out_hbm, lab_v, rows_v, trans_v, off_v, sem):
    wid = lax.axis_index("s") * NC + lax.axis_index("c")

    def init_off(i, c):
        p = lax.iota(jnp.int32, LANES) + i * LANES
        off_v[pl.ds(i * LANES, LANES)] = jnp.where(p < P, (p // L) * VOCAB, 0)
        return c
    lax.fori_loop(0, PPAD // LANES, init_off, 0)

    lab_v[pl.ds(PPAD - LANES, LANES)] = jnp.zeros((LANES,), jnp.int32)

    def per_b(i, c):
        b = wid * BPW + i
        pltpu.sync_copy(labels_hbm.at[b], lab_v.at[pl.ds(0, P)])

        def add_off(j, cc):
            sl = pl.ds(j * LANES, LANES)
            lab_v[sl] = lab_v[sl] + off_v[sl]
            return cc
        lax.fori_loop(0, PPAD // LANES, add_off, 0)

        pltpu.async_copy(tables_hbm.at[lab_v], rows_v, sem).wait()

        def per_l(l, cc):
            lane = lax.iota(jnp.int32, LANES) * L
            for h in range(D // LANES):
                a0 = rows_v[l, pl.ds(h * LANES, LANES)]
                a1 = rows_v[L + l, pl.ds(h * LANES, LANES)]
                for f in range(2, F, 2):
                    a0 = a0 + rows_v[f * L + l, pl.ds(h * LANES, LANES)]
                    a1 = a1 + rows_v[(f + 1) * L + l, pl.ds(h * LANES, LANES)]
                acc = (a0 + a1) * (1.0 / F)
                plsc.store_scatter(trans_v, [lane + (h * LANES * L + l)], acc)
            return cc
        lax.fori_loop(0, L, per_l, 0)

        pltpu.sync_copy(trans_v, out_hbm.at[b])
        return c
    lax.fori_loop(0, BPW, per_b, 0)


@jax.jit
def kernel(labels, tables):
    labels32 = labels.astype(jnp.int32).reshape(B, P)
    tables_flat = tables.reshape(F * VOCAB, D)
    k = pl.kernel(
        _body,
        out_type=jax.ShapeDtypeStruct((B, D * L), jnp.float32),
        mesh=plsc.VectorSubcoreMesh(core_axis_name="c", subcore_axis_name="s"),
        scratch_types=[
            pltpu.VMEM((PPAD,), jnp.int32),
            pltpu.VMEM((PPAD, D), jnp.float32),
            pltpu.VMEM((D * L,), jnp.float32),
            pltpu.VMEM((PPAD,), jnp.int32),
            pltpu.SemaphoreType.DMA,
        ],
    )
    out = k(labels32, tables_flat)
    return out.reshape(B, D, L)


# speedup vs baseline: 10.6145x; 10.6145x over previous
import numpy as np

import jax
import jax.numpy as jnp
from jax import lax
from jax.experimental import pallas as pl
from jax.experimental.pallas import tpu as pltpu
from jax.experimental.pallas import tpu_sc as plsc

B = 4096
F = 26
L = 50
VOCAB = 100000
D = 32

NC = 2
NS = 16
NW = NC * NS
BPW = B // NW
P = F * L
LANES = 16
PPAD = ((P + LANES - 1) // LANES) * LANES
PROW = P + 4


def _body(labels_hbm, tables_hbm, off_hbm, out_hbm,
          lab_v, rows_v, trans_v, off_v, sem):
    wid = lax.axis_index("s") * NC + lax.axis_index("c")

    pltpu.sync_copy(off_hbm, off_v)
    lab_v[pl.ds(PPAD - LANES, LANES)] = jnp.zeros((LANES,), jnp.int32)

    def per_b(i, c):
        b = wid * BPW + i
        pltpu.sync_copy(labels_hbm.at[pl.ds(b * PROW, PROW)],
                        lab_v.at[pl.ds(0, PROW)])

        def add_off(j, cc):
            sl = pl.ds(j * LANES, LANES)
            lab_v[sl] = lab_v[sl] + off_v[sl]
            return cc
        lax.fori_loop(0, PPAD // LANES, add_off, 0)

        pltpu.async_copy(tables_hbm.at[lab_v], rows_v, sem).wait()

        def per_l(l, cc):
            lane = lax.iota(jnp.int32, LANES) * L
            for h in range(D // LANES):
                a0 = rows_v[l, pl.ds(h * LANES, LANES)]
                a1 = rows_v[L + l, pl.ds(h * LANES, LANES)]
                for f in range(2, F, 2):
                    a0 = a0 + rows_v[f * L + l, pl.ds(h * LANES, LANES)]
                    a1 = a1 + rows_v[(f + 1) * L + l, pl.ds(h * LANES, LANES)]
                acc = (a0 + a1) * (1.0 / F)
                plsc.store_scatter(trans_v, [lane + (h * LANES * L + l)], acc)
            return cc
        lax.fori_loop(0, L, per_l, 0)

        pltpu.sync_copy(trans_v, out_hbm.at[pl.ds(b * (D * L), D * L)])
        return c
    lax.fori_loop(0, BPW, per_b, 0)


_OFF_NP = np.zeros((PPAD,), np.int32)
_OFF_NP[:P] = np.repeat(np.arange(F, dtype=np.int32) * VOCAB, L)


@jax.jit
def kernel(labels, tables):
    labels32 = jnp.pad(labels.astype(jnp.int32).reshape(B, P),
                       ((0, 0), (0, PROW - P))).reshape(B * PROW)
    tables_flat = tables.reshape(F * VOCAB, D)
    off = jnp.asarray(_OFF_NP)
    k = pl.kernel(
        _body,
        out_type=jax.ShapeDtypeStruct((B * D * L,), jnp.float32),
        mesh=plsc.VectorSubcoreMesh(core_axis_name="c", subcore_axis_name="s"),
        compiler_params=pltpu.CompilerParams(use_tc_tiling_on_sc=False, needs_layout_passes=False),
        scratch_types=[
            pltpu.VMEM((PPAD,), jnp.int32),
            pltpu.VMEM((PPAD, D), jnp.float32),
            pltpu.VMEM((D * L,), jnp.float32),
            pltpu.VMEM((PPAD,), jnp.int32),
            pltpu.SemaphoreType.DMA,
        ],
    )
    out = k(labels32, tables_flat, off)
    return out.reshape(B, D, L)
